# baseline (speedup 1.0000x reference)
import functools

import jax
import jax.numpy as jnp
from jax import lax
from jax.experimental import pallas as pl
from jax.experimental.pallas import tpu as pltpu
from jax.experimental.pallas import tpu_sc as plsc

B, S, D = 4, 4096, 1024
SC_S = 1024
TC_S = S - SC_S
NC, NS = 2, 16
NW = NC * NS
ROWS_PER_W = SC_S // NW
R = 16
CHUNKS = ROWS_PER_W // R
NBUF = 4
T = CHUNKS * B

_mesh = plsc.VectorSubcoreMesh(core_axis_name="c", subcore_axis_name="s")


@functools.partial(
    pl.kernel,
    out_type=jax.ShapeDtypeStruct((B, SC_S, D), jnp.float32),
    mesh=_mesh,
    scratch_types=[
        pltpu.VMEM((2, R, D), jnp.float32),
        pltpu.VMEM((NBUF, R, D), jnp.float32),
        pltpu.SemaphoreType.DMA((2,)),
        pltpu.SemaphoreType.DMA((NBUF,)),
        pltpu.SemaphoreType.DMA((NBUF,)),
    ],
)
def _sc_add(in_hbm, emb_hbm, out_hbm, emb_v, buf_v, emb_sem, in_sem, out_sem):
    wid = lax.axis_index("s") * NC + lax.axis_index("c")
    row_base = wid * ROWS_PER_W

    def emb_copy(c):
        return pltpu.make_async_copy(
            emb_hbm.at[pl.ds(TC_S + row_base + c * R, R)],
            emb_v.at[c % 2], emb_sem.at[c % 2])

    def in_copy(t):
        c, b = divmod(t, B)
        return pltpu.make_async_copy(
            in_hbm.at[b, pl.ds(TC_S + row_base + c * R, R)],
            buf_v.at[t % NBUF], in_sem.at[t % NBUF])

    def out_copy(t):
        c, b = divmod(t, B)
        return pltpu.make_async_copy(
            buf_v.at[t % NBUF],
            out_hbm.at[b, pl.ds(row_base + c * R, R)],
            out_sem.at[t % NBUF])

    emb_copy(0).start()
    in_copy(0).start()
    in_copy(1).start()

    for t in range(T):
        c, b = divmod(t, B)
        if b == 0:
            emb_copy(c).wait()
            if c + 1 < CHUNKS:
                emb_copy(c + 1).start()
        in_copy(t).wait()
        if t + 2 < T:
            if t - 2 >= 0:
                out_copy(t - 2).wait()
            in_copy(t + 2).start()

        buf = buf_v.at[t % NBUF]
        emb = emb_v.at[c % 2]

        @plsc.parallel_loop(0, D, step=16)
        def add_body(o):
            for r in range(R):
                plsc.addupdate(buf.at[r, pl.ds(o, 16)], emb[r, pl.ds(o, 16)])

        out_copy(t).start()

    out_copy(T - 2).wait()
    out_copy(T - 1).wait()


BLK = 256

def _tc_body(in_ref, emb_ref, out_ref):
    out_ref[...] = in_ref[...] + emb_ref[None]


_tc_add = pl.pallas_call(
    _tc_body,
    out_shape=jax.ShapeDtypeStruct((B, TC_S, D), jnp.float32),
    grid=(TC_S // BLK,),
    in_specs=[
        pl.BlockSpec((B, BLK, D), lambda s: (0, s, 0)),
        pl.BlockSpec((BLK, D), lambda s: (s, 0)),
    ],
    out_specs=pl.BlockSpec((B, BLK, D), lambda s: (0, s, 0)),
)


def kernel(inputs, embedding):
    sc = _sc_add(inputs, embedding)
    tc = _tc_add(inputs, embedding)
    return tc, sc


# speedup vs baseline: 1.1903x; 1.1903x over previous
import jax
import jax.numpy as jnp
from jax.experimental import pallas as pl
from jax.experimental.pallas import tpu as pltpu

B, S, D = 4, 4096, 1024
BLK = 512


def _body(in_ref, emb_ref, out_ref):
    out_ref[...] = in_ref[...] + emb_ref[...][None]


_add = pl.pallas_call(
    _body,
    out_shape=jax.ShapeDtypeStruct((B, S, D), jnp.float32),
    grid=(S // BLK, B),
    in_specs=[
        pl.BlockSpec((1, BLK, D), lambda s, b: (b, s, 0)),
        pl.BlockSpec((BLK, D), lambda s, b: (s, 0)),
    ],
    out_specs=pl.BlockSpec((1, BLK, D), lambda s, b: (b, s, 0)),
    compiler_params=pltpu.CompilerParams(
        dimension_semantics=("arbitrary", "arbitrary"),
    ),
)


def kernel(inputs, embedding):
    return _add(inputs, embedding)


# speedup vs baseline: 1.3750x; 1.1552x over previous
import jax
import jax.numpy as jnp
from jax.experimental import pallas as pl
from jax.experimental.pallas import tpu as pltpu

B, S, D = 4, 4096, 1024
BLK = 512


def _body(in_ref, emb_ref, out_ref):
    out_ref[...] = in_ref[...] + emb_ref[...][None]


_add = pl.pallas_call(
    _body,
    out_shape=jax.ShapeDtypeStruct((B, S, D), jnp.float32),
    grid=(S // BLK,),
    in_specs=[
        pl.BlockSpec((B, BLK, D), lambda s: (0, s, 0)),
        pl.BlockSpec((BLK, D), lambda s: (s, 0)),
    ],
    out_specs=pl.BlockSpec((B, BLK, D), lambda s: (0, s, 0)),
    compiler_params=pltpu.CompilerParams(
        dimension_semantics=("arbitrary",),
    ),
)


def kernel(inputs, embedding):
    return _add(inputs, embedding)
